# Initial kernel scaffold; baseline (speedup 1.0000x reference)
import jax
import jax.numpy as jnp
from jax.experimental import pallas as pl


def kernel(x, x_knn, knn_idx, p_r, W1, b1, Wx, bx, Wp1, bn_gamma, bn_beta, Wp2, bp2):
    raise NotImplementedError("write your pallas kernel here")


# speedup vs baseline: 3.3687x; 3.3687x over previous
import functools

import jax
import jax.numpy as jnp
from jax import lax
from jax.experimental import pallas as pl
from jax.experimental.pallas import tpu as pltpu
from jax.experimental.pallas import tpu_sc as plsc

N = 10000
K = 32
C = 128
CS = 16
NK = N * K

NC = 2
NS = 16
GRP = 128
NGRP = NK // GRP
SB = 4
HALF = N // NC
DUMP = 256
TROWS = HALF + DUMP
ZPT = 328
DPT = 312

BP = 8000
BE = 2560
BN_ = 2000


def _moments_body(pr_ref, wp1_ref, out_ref):
    i = pl.program_id(0)
    p = pr_ref[...]
    w = wp1_ref[...]
    h = (p[:, 0:1] * w[0:1, :] + p[:, 1:2] * w[1:2, :] + p[:, 2:3] * w[2:3, :])
    s1 = jnp.sum(h, axis=0, keepdims=True)
    s2 = jnp.sum(h * h, axis=0, keepdims=True)
    upd = jnp.concatenate([s1, s2], axis=0)

    @pl.when(i == 0)
    def _():
        out_ref[...] = jnp.zeros_like(out_ref)

    out_ref[...] += upd


def _moments(pr, Wp1):
    return pl.pallas_call(
        _moments_body,
        grid=(NK // BP,),
        in_specs=[
            pl.BlockSpec((BP, 3), lambda i: (i, 0)),
            pl.BlockSpec((3, 3), lambda i: (0, 0)),
        ],
        out_specs=pl.BlockSpec((2, 3), lambda i: (0, 0)),
        out_shape=jax.ShapeDtypeStruct((2, 3), jnp.float32),
    )(pr, Wp1)


def _dense_body(xk_ref, pr_ref, wp1_ref, scale_ref, shift_ref, wp2_ref,
                w1a_ref, bp2_ref, b1_ref, w1b_ref, wx_ref, bx_ref,
                s_ref, xv_ref, macc_ref):
    i = pl.program_id(0)
    wf = wp1_ref[...] * scale_ref[...]
    wc = jnp.dot(wp2_ref[...], w1a_ref[...],
                 preferred_element_type=jnp.float32)
    c0 = jnp.dot(bp2_ref[...], w1a_ref[...],
                 preferred_element_type=jnp.float32) + b1_ref[...]

    p = pr_ref[...]
    h = (p[:, 0:1] * wf[0:1, :] + p[:, 1:2] * wf[1:2, :]
         + p[:, 2:3] * wf[2:3, :] + shift_ref[...])
    hr = jnp.maximum(h, 0.0)
    sh = (hr[:, 0:1] * wc[0:1, :] + hr[:, 1:2] * wc[1:2, :]
          + hr[:, 2:3] * wc[2:3, :] + c0)

    xk = xk_ref[...]
    z1 = jnp.dot(xk, w1b_ref[...], preferred_element_type=jnp.float32)
    z2 = jnp.dot(xk, wx_ref[...], preferred_element_type=jnp.float32)
    s = jnp.maximum(z1 + sh, 0.0)
    xv = jnp.maximum(z2 + bx_ref[...], 0.0)
    s_ref[...] = s
    xv_ref[...] = xv

    @pl.when(i == 0)
    def _():
        macc_ref[...] = jnp.zeros_like(macc_ref)

    macc_ref[...] = jnp.maximum(macc_ref[...], jnp.full((8, 128), jnp.max(s)))


def _dense(xk, pr, Wp1, scale, shift, Wp2, W1a, bp2, b1, W1b, Wx, bx):
    full = lambda shape: pl.BlockSpec(shape, lambda i: tuple(0 for _ in shape))
    return pl.pallas_call(
        _dense_body,
        grid=(NK // BE,),
        in_specs=[
            pl.BlockSpec((BE, 128), lambda i: (i, 0)),
            pl.BlockSpec((BE, 3), lambda i: (i, 0)),
            full((3, 3)), full((1, 3)), full((1, 3)),
            full((3, 128)), full((128, 16)), full((1, 128)), full((1, 16)),
            full((128, 16)), full((128, 16)), full((1, 16)),
        ],
        out_specs=[
            pl.BlockSpec((BE, 16), lambda i: (i, 0)),
            pl.BlockSpec((BE, 16), lambda i: (i, 0)),
            pl.BlockSpec((8, 128), lambda i: (0, 0)),
        ],
        out_shape=[
            jax.ShapeDtypeStruct((NK, 16), jnp.float32),
            jax.ShapeDtypeStruct((NK, 16), jnp.float32),
            jax.ShapeDtypeStruct((8, 128), jnp.float32),
        ],
    )(xk, pr, Wp1, scale, shift, Wp2, W1a, bp2, b1, W1b, Wx, bx)


def _payload_body(s_ref, xv_ref, m_ref, e_ref, w_ref):
    e = jnp.exp(s_ref[...] - m_ref[...])
    e_ref[...] = e
    w_ref[...] = xv_ref[...] * e


def _payload(s_e, xv_e, m):
    spec16 = pl.BlockSpec((BE, 16), lambda i: (i, 0))
    return pl.pallas_call(
        _payload_body,
        grid=(NK // BE,),
        in_specs=[spec16, spec16, pl.BlockSpec((1, 16), lambda i: (0, 0))],
        out_specs=[spec16, spec16],
        out_shape=[jax.ShapeDtypeStruct((NK, 16), jnp.float32)] * 2,
    )(s_e, xv_e, m)


def _sc_table_pass(pay_hbm, idx_hbm, z_hbm, out_hbm, pay_v, i_v, acc,
                   qidx, base, sid):
    pltpu.sync_copy(z_hbm.at[pl.ds(sid * ZPT, ZPT)],
                    acc.at[pl.ds(sid * ZPT, ZPT)])

    @pl.when(sid == 0)
    def _():
        pltpu.sync_copy(z_hbm.at[pl.ds(TROWS - 8, 8)],
                        acc.at[pl.ds(TROWS - 8, 8)])

    plsc.subcore_barrier()

    def remap(j):
        for k in range(GRP // 16):
            v = i_v[j, pl.ds(k * 16, 16)]
            local = v - base
            ok = (local >= 0) & (local < HALF)
            i_v[j, pl.ds(k * 16, 16)] = jnp.where(
                ok, local, HALF + (v & (DUMP - 1)))

    start = jnp.where(sid < 8, sid * 20, 160 + (sid - 8) * 19)
    cnt = jnp.where(sid < 8, 20, 19)

    def superchunk(t, _):
        pltpu.sync_copy(idx_hbm.at[pl.ds(t * 8, 8)], i_v)
        for j in range(8):
            remap(j)
        for hh in range(2):
            pltpu.sync_copy(
                pay_hbm.at[pl.ds((t * 8 + hh * 4) * GRP, SB * GRP)], pay_v)
            for j in range(SB):
                pltpu.sync_copy(pay_v.at[pl.ds(j * GRP, GRP)],
                                acc.at[i_v.at[hh * SB + j]], add=True)
        return 0

    lax.fori_loop(start, start + cnt, superchunk, 0)

    @pl.when(sid == 15)
    def _():
        pltpu.sync_copy(idx_hbm.at[pl.ds(NGRP - 4, 4)], i_v.at[pl.ds(0, 4)])
        for j in range(4):
            remap(j)
        pltpu.sync_copy(pay_hbm.at[pl.ds((NGRP - 4) * GRP, SB * GRP)], pay_v)
        for j in range(4):
            pltpu.sync_copy(pay_v.at[pl.ds(j * GRP, GRP)],
                            acc.at[i_v.at[j]], add=True)

    plsc.subcore_barrier()

    pltpu.sync_copy(acc.at[pl.ds(sid * DPT, DPT)],
                    out_hbm.at[qidx, pl.ds(sid * DPT, DPT)])

    @pl.when(sid == 0)
    def _():
        pltpu.sync_copy(acc.at[pl.ds(HALF - 8, 8)],
                        out_hbm.at[qidx, pl.ds(HALF - 8, 8)])

    plsc.subcore_barrier()


def _sc_body(e_hbm, w_hbm, idx_hbm, z_hbm, out_hbm, pay_v, i_v, acc):
    cid = lax.axis_index("c")
    sid = lax.axis_index("s")
    for c in range(NC):
        @pl.when(cid == c)
        def _(c=c):
            for p, pay in enumerate([e_hbm, w_hbm]):
                _sc_table_pass(pay, idx_hbm, z_hbm, out_hbm, pay_v, i_v,
                               acc, c * 2 + p, c * HALF, sid)


@functools.cache
def _sc_scatter():
    return pl.kernel(
        _sc_body,
        out_type=jax.ShapeDtypeStruct((4, HALF, 16), jnp.float32),
        mesh=plsc.VectorSubcoreMesh(core_axis_name="c", subcore_axis_name="s",
                                    num_cores=NC, num_subcores=NS),
        scratch_types=[
            pltpu.VMEM((SB * GRP, 16), jnp.float32),
            pltpu.VMEM((8, GRP), jnp.int32),
            pltpu.VMEM_SHARED((TROWS, 16), jnp.float32),
        ],
    )


def _final_body(x_ref, e_ref, w_ref, out_ref):
    den = e_ref[...]
    num = w_ref[...]
    r = jnp.where(den != 0.0, num / den, 0.0)
    out_ref[...] = x_ref[...] + jnp.concatenate([r] * 8, axis=1)


def _final(x, e, w):
    spec16 = pl.BlockSpec((BN_, 16), lambda i: (i, 0))
    return pl.pallas_call(
        _final_body,
        grid=(N // BN_,),
        in_specs=[pl.BlockSpec((BN_, 128), lambda i: (i, 0)), spec16, spec16],
        out_specs=pl.BlockSpec((BN_, 128), lambda i: (i, 0)),
        out_shape=jax.ShapeDtypeStruct((N, 128), jnp.float32),
    )(x, e, w)


def kernel(x, x_knn, knn_idx, p_r, W1, b1, Wx, bx, Wp1, bn_gamma, bn_beta,
           Wp2, bp2):
    xk = x_knn.reshape(NK, C)
    pr = p_r.reshape(NK, 3)
    idx2d = knn_idx.reshape(NGRP, GRP).astype(jnp.int32)

    mom = _moments(pr, Wp1)
    mean = mom[0] / NK
    var = mom[1] / NK - mean * mean
    scale = (bn_gamma * lax.rsqrt(var + 1e-5)).reshape(1, 3)
    shift = (bn_beta - mean * scale[0]).reshape(1, 3)

    s_e, xv_e, macc = _dense(
        xk, pr, Wp1, scale, shift, Wp2, W1[:C], bp2.reshape(1, C),
        b1.reshape(1, CS), W1[C:], Wx, bx.reshape(1, CS))

    e_pay, w_pay = _payload(s_e, xv_e, macc[0:1, 0:16])

    zeros16 = jnp.zeros((TROWS, 16), jnp.float32)
    acc = _sc_scatter()(e_pay, w_pay, idx2d, zeros16)
    e_full = jnp.concatenate([acc[0], acc[2]], axis=0)
    w_full = jnp.concatenate([acc[1], acc[3]], axis=0)
    return _final(x, e_full, w_full)
